# Initial kernel scaffold; baseline (speedup 1.0000x reference)
import jax
import jax.numpy as jnp
from jax.experimental import pallas as pl


def kernel(pts0, pts1, pts2, pts3, feats0, feats1, feats2, feats3, fp3_W1, fp3_b1, fp3_g1, fp3_be1, fp3_W2, fp3_b2, fp3_g2, fp3_be2, fp2_W1, fp2_b1, fp2_g1, fp2_be1, fp2_W2, fp2_b2, fp2_g2, fp2_be2, fp1_W1, fp1_b1, fp1_g1, fp1_be1, fp1_W2, fp1_b2, fp1_g2, fp1_be2):
    raise NotImplementedError("write your pallas kernel here")


# speedup vs baseline: 12.2231x; 12.2231x over previous
import functools

import jax
import jax.numpy as jnp
from jax.experimental import pallas as pl


_EPS_DIST = 1e-8
_EPS_BN = 1e-5


def _mm1_tail(wmat, p2_ref, p1_ref, w1_ref, y_ref, st_ref):
    interp = jax.lax.dot_general(
        p2_ref[0], wmat, (((1,), (1,)), ((), ())),
        preferred_element_type=jnp.float32,
        precision=jax.lax.Precision.HIGHEST)
    x = jnp.concatenate([p1_ref[0], interp], axis=0)
    y = jax.lax.dot_general(w1_ref[...], x, (((1,), (0,)), ((), ())),
                            preferred_element_type=jnp.float32)
    y_ref[0] = y

    first = jnp.logical_and(pl.program_id(0) == 0, pl.program_id(1) == 0)
    st = jnp.stack([jnp.sum(y, axis=1), jnp.sum(y * y, axis=1)], axis=1)

    @pl.when(first)
    def _():
        st_ref[...] = st

    @pl.when(jnp.logical_not(first))
    def _():
        st_ref[...] = st_ref[...] + st


def _knn_mm1_body(nd_ref, p2_ref, p1_ref, w1_ref, y_ref, st_ref, *, S):
    nd = nd_ref[0]
    nb = nd.shape[0]
    iota = jax.lax.broadcasted_iota(jnp.int32, (nb, S), 1)
    wmat = jnp.zeros((nb, S), jnp.float32)
    norm = jnp.zeros((nb,), jnp.float32)
    recips = []
    ohs = []
    for _ in range(3):
        m = jnp.max(nd, axis=1)
        amax = jnp.min(jnp.where(nd == m[:, None], iota, S), axis=1)
        oh = iota == amax[:, None]
        recip = 1.0 / (-m + _EPS_DIST)
        norm = norm + recip
        recips.append(recip)
        ohs.append(oh)
        nd = jnp.where(oh, -jnp.inf, nd)
    inv_norm = 1.0 / norm
    for oh, recip in zip(ohs, recips):
        w = recip * inv_norm
        wmat = wmat + jnp.where(oh, w[:, None], 0.0)
    _mm1_tail(wmat, p2_ref, p1_ref, w1_ref, y_ref, st_ref)


def _topk_mm1_body(nv_ref, ni_ref, p2_ref, p1_ref, w1_ref, y_ref, st_ref, *,
                   S):
    nv = nv_ref[0]
    ni = ni_ref[0]
    nb = nv.shape[0]
    iota = jax.lax.broadcasted_iota(jnp.int32, (nb, S), 1)
    recips = [1.0 / (-nv[:, k] + _EPS_DIST) for k in range(3)]
    inv_norm = 1.0 / (recips[0] + recips[1] + recips[2])
    wmat = jnp.zeros((nb, S), jnp.float32)
    for k in range(3):
        w = recips[k] * inv_norm
        wmat = wmat + jnp.where(iota == ni[:, k][:, None], w[:, None], 0.0)
    _mm1_tail(wmat, p2_ref, p1_ref, w1_ref, y_ref, st_ref)


def _mm1_specs(B, N, S, C1, D1, D2, nb, sel_specs):
    return dict(
        grid=(B, N // nb),
        in_specs=sel_specs + [
            pl.BlockSpec((1, D2, S), lambda b, n: (b, 0, 0)),
            pl.BlockSpec((1, D1, nb), lambda b, n: (b, 0, n)),
            pl.BlockSpec((C1, D1 + D2), lambda b, n: (0, 0)),
        ],
        out_specs=[
            pl.BlockSpec((1, C1, nb), lambda b, n: (b, 0, n)),
            pl.BlockSpec((C1, 2), lambda b, n: (0, 0)),
        ],
        out_shape=[
            jax.ShapeDtypeStruct((B, C1, N), jnp.float32),
            jax.ShapeDtypeStruct((C1, 2), jnp.float32),
        ],
    )


def _knn_mm1(negd, p2, p1, w1, nb):
    B, N, S = negd.shape
    D2 = p2.shape[1]
    D1 = p1.shape[1]
    C1 = w1.shape[0]
    sel = [pl.BlockSpec((1, nb, S), lambda b, n: (b, n, 0))]
    y, st = pl.pallas_call(
        functools.partial(_knn_mm1_body, S=S),
        **_mm1_specs(B, N, S, C1, D1, D2, nb, sel),
    )(negd, p2, p1, w1)
    return y, st


def _topk_mm1(nv, ni, S, p2, p1, w1, nb):
    B, N, _ = nv.shape
    D2 = p2.shape[1]
    D1 = p1.shape[1]
    C1 = w1.shape[0]
    sel = [pl.BlockSpec((1, nb, 3), lambda b, n: (b, n, 0)),
           pl.BlockSpec((1, nb, 3), lambda b, n: (b, n, 0))]
    y, st = pl.pallas_call(
        functools.partial(_topk_mm1_body, S=S),
        **_mm1_specs(B, N, S, C1, D1, D2, nb, sel),
    )(nv, ni, p2, p1, w1)
    return y, st


def _affine_mm_body(y_ref, sc_ref, w_ref, o_ref, st_ref, *, with_stats):
    a = sc_ref[:, 0:1]
    c = sc_ref[:, 1:2]
    z = jnp.maximum(y_ref[0] * a + c, 0.0)
    o = jax.lax.dot_general(w_ref[...], z, (((1,), (0,)), ((), ())),
                            preferred_element_type=jnp.float32)
    o_ref[0] = o
    if with_stats:
        first = jnp.logical_and(pl.program_id(0) == 0, pl.program_id(1) == 0)
        st = jnp.stack([jnp.sum(o, axis=1), jnp.sum(o * o, axis=1)], axis=1)

        @pl.when(first)
        def _():
            st_ref[...] = st

        @pl.when(jnp.logical_not(first))
        def _():
            st_ref[...] = st_ref[...] + st


def _affine_mm(y, sc, w, nb, with_stats=True):
    B, C1, N = y.shape
    C2 = w.shape[0]
    o, st = pl.pallas_call(
        functools.partial(_affine_mm_body, with_stats=with_stats),
        grid=(B, N // nb),
        in_specs=[
            pl.BlockSpec((1, C1, nb), lambda b, n: (b, 0, n)),
            pl.BlockSpec((C1, 2), lambda b, n: (0, 0)),
            pl.BlockSpec((C2, C1), lambda b, n: (0, 0)),
        ],
        out_specs=[
            pl.BlockSpec((1, C2, nb), lambda b, n: (b, 0, n)),
            pl.BlockSpec((C2, 2), lambda b, n: (0, 0)),
        ],
        out_shape=[
            jax.ShapeDtypeStruct((B, C2, N), jnp.float32),
            jax.ShapeDtypeStruct((C2, 2), jnp.float32),
        ],
    )(y, sc, w)
    return o, st


def _final_body(y_ref, sc_ref, o_ref):
    a = sc_ref[:, 0:1]
    c = sc_ref[:, 1:2]
    o_ref[0] = jnp.maximum(y_ref[0] * a + c, 0.0)


def _final(y, sc, nb):
    B, C2, N = y.shape
    return pl.pallas_call(
        _final_body,
        grid=(B, N // nb),
        in_specs=[
            pl.BlockSpec((1, C2, nb), lambda b, n: (b, 0, n)),
            pl.BlockSpec((C2, 2), lambda b, n: (0, 0)),
        ],
        out_specs=pl.BlockSpec((1, C2, nb), lambda b, n: (b, 0, n)),
        out_shape=jax.ShapeDtypeStruct((B, C2, N), jnp.float32),
    )(y, sc)


def _bn_scale_shift(st, count, gamma, beta):
    mean = st[:, 0] / count
    var = st[:, 1] / count - mean * mean
    a = gamma * jax.lax.rsqrt(var + _EPS_BN)
    c = beta - a * mean
    return jnp.stack([a, c], axis=1)


def kernel(pts0, pts1, pts2, pts3, feats0, feats1, feats2, feats3,
           fp3_W1, fp3_b1, fp3_g1, fp3_be1, fp3_W2, fp3_b2, fp3_g2, fp3_be2,
           fp2_W1, fp2_b1, fp2_g1, fp2_be1, fp2_W2, fp2_b2, fp2_g2, fp2_be2,
           fp1_W1, fp1_b1, fp1_g1, fp1_be1, fp1_W2, fp1_b2, fp1_g2, fp1_be2):
    B = pts0.shape[0]

    def _negd(xyz1, xyz2):
        x1 = jnp.transpose(xyz1, (0, 2, 1))
        x2 = jnp.transpose(xyz2, (0, 2, 1))
        d = (jnp.sum(x1 * x1, axis=-1)[:, :, None]
             + jnp.sum(x2 * x2, axis=-1)[:, None, :]
             - 2.0 * jnp.einsum('bnc,bsc->bns', x1, x2))
        return -d

    nv3, ni3 = jax.lax.top_k(_negd(pts2, pts3), 3)
    nd2 = _negd(pts1, pts2)
    nd1 = _negd(pts0, pts1)

    y1, st = _topk_mm1(nv3, ni3, 64, feats3, feats2, fp3_W1, nb=256)
    sc = _bn_scale_shift(st, B * 256, fp3_g1, fp3_be1)
    y2, st = _affine_mm(y1, sc, fp3_W2, nb=256)
    sc = _bn_scale_shift(st, B * 256, fp3_g2, fp3_be2)
    l2 = _final(y2, sc, nb=256)

    y1, st = _knn_mm1(nd2, l2, feats1, fp2_W1, nb=512)
    sc = _bn_scale_shift(st, B * 1024, fp2_g1, fp2_be1)
    y2, st = _affine_mm(y1, sc, fp2_W2, nb=512)
    sc = _bn_scale_shift(st, B * 1024, fp2_g2, fp2_be2)
    l1 = _final(y2, sc, nb=512)

    p1 = jnp.concatenate([pts0, feats0], axis=1)
    y1, st = _knn_mm1(nd1, l1, p1, fp1_W1, nb=512)
    sc = _bn_scale_shift(st, B * 4096, fp1_g1, fp1_be1)
    y2, st = _affine_mm(y1, sc, fp1_W2, nb=512)
    sc = _bn_scale_shift(st, B * 4096, fp1_g2, fp1_be2)
    return _final(y2, sc, nb=512)
